# Initial kernel scaffold; baseline (speedup 1.0000x reference)
import jax
import jax.numpy as jnp
from jax.experimental import pallas as pl


def kernel(x, h, edge_src, edge_dst, edges_r_ij, edges_e_ij, W1, b1, W2, b2, Wp1, bp1, Wp2, bp2, W_ih, W_hh, b_ih, b_hh):
    raise NotImplementedError("write your pallas kernel here")


# speedup vs baseline: 1.5576x; 1.5576x over previous
import functools

import jax
import jax.numpy as jnp
from jax import lax
from jax.experimental import pallas as pl
from jax.experimental.pallas import tpu as pltpu
from jax.experimental.pallas import tpu_sc as plsc

F32 = jnp.float32


def _silu(x):
    return x * jax.nn.sigmoid(x)


def _proj_body(h_ref, w1a_ref, w1b_ref, ga_ref, gb_ref):
    hb = h_ref[...]
    ga_ref[...] = jnp.dot(hb, w1a_ref[...], preferred_element_type=F32)
    gb_ref[...] = jnp.dot(hb, w1b_ref[...], preferred_element_type=F32)


def _edge_body(a_ref, b_ref, r_ref, w1c_ref, b1_ref, w2_ref, b2_ref,
               wp1_ref, bp1_ref, wp2_ref, bp2_ref, m_ref, w_ref):
    pre = a_ref[...] + b_ref[...] + r_ref[...] * w1c_ref[...] + b1_ref[...]
    t1 = _silu(pre)
    mm = _silu(jnp.dot(t1, w2_ref[...], preferred_element_type=F32) + b2_ref[...])
    m_ref[...] = mm
    u = _silu(jnp.dot(mm, wp1_ref[...], preferred_element_type=F32) + bp1_ref[...])
    w_ref[...] = jnp.dot(u, wp2_ref[...], preferred_element_type=F32) + bp2_ref[...]


def _node_body(pm_ref, cnt_ref, h_ref, wih_ref, whh_ref,
               bih_ref, bhh_ref, hp_ref):
    sm = pm_ref[0] + pm_ref[1]
    m_i = sm / jnp.maximum(cnt_ref[...], 1.0)
    h = h_ref[...]
    gi = jnp.dot(m_i, wih_ref[...], preferred_element_type=F32) + bih_ref[...]
    gh = jnp.dot(h, whh_ref[...], preferred_element_type=F32) + bhh_ref[...]
    r = jax.nn.sigmoid(gi[:, 0:128] + gh[:, 0:128])
    z = jax.nn.sigmoid(gi[:, 128:256] + gh[:, 128:256])
    n = jnp.tanh(gi[:, 256:384] + r * gh[:, 256:384])
    hp_ref[...] = (1.0 - z) * n + z * h


_NC = 2
_NS = 16
_NW = _NC * _NS
_CH = 80


def _make_sc_gather(n, e, f):
    ew_per = e // _NW
    niter = ew_per // _CH
    mesh = plsc.VectorSubcoreMesh(core_axis_name="c", subcore_axis_name="s")

    @functools.partial(
        pl.kernel,
        out_type=(jax.ShapeDtypeStruct((e, f), F32),
                  jax.ShapeDtypeStruct((e, f), F32)),
        mesh=mesh,
        scratch_types=[
            pltpu.VMEM((_CH,), jnp.int32),
            pltpu.VMEM((_CH,), jnp.int32),
            pltpu.VMEM((_CH, f), F32),
            pltpu.VMEM((_CH, f), F32),
            pltpu.SemaphoreType.DMA,
            pltpu.SemaphoreType.DMA,
        ],
    )
    def sc_gather(ga_hbm, gb_hbm, src_hbm, dst_hbm, outa_hbm, outb_hbm,
                  idxs_v, idxd_v, rowa_v, rowb_v, sem_a, sem_b):
        wid = lax.axis_index("s") * _NC + lax.axis_index("c")
        base = wid * ew_per

        def body(i, carry):
            off = base + i * _CH
            pltpu.sync_copy(src_hbm.at[pl.ds(off, _CH)], idxs_v)
            pltpu.sync_copy(dst_hbm.at[pl.ds(off, _CH)], idxd_v)
            cpa = pltpu.async_copy(ga_hbm.at[idxs_v], rowa_v, sem_a)
            cpb = pltpu.async_copy(gb_hbm.at[idxd_v], rowb_v, sem_b)
            cpa.wait()
            cpb.wait()
            pltpu.sync_copy(rowa_v, outa_hbm.at[pl.ds(off, _CH)])
            pltpu.sync_copy(rowb_v, outb_hbm.at[pl.ds(off, _CH)])
            return carry

        lax.fori_loop(0, niter, body, 0)

    return sc_gather


def _make_sc_scatter(n_pad, e, f):
    ew_per = e // _NW
    niter = ew_per // _CH
    npt = n_pad // _NS
    nzc = npt // _CH
    mesh = plsc.VectorSubcoreMesh(core_axis_name="c", subcore_axis_name="s")

    @functools.partial(
        pl.kernel,
        out_type=jax.ShapeDtypeStruct((_NC, n_pad, f), F32),
        mesh=mesh,
        scratch_types=[
            pltpu.VMEM((_CH,), jnp.int32),
            pltpu.VMEM((_CH, f), F32),
            pltpu.VMEM_SHARED((n_pad, f), F32),
        ],
    )
    def sc_scatter(m_hbm, src_hbm, zm_hbm, outm_hbm, idx_v, mrow_v, accm):
        c = lax.axis_index("c")
        sid = lax.axis_index("s")
        row0 = sid * npt
        pltpu.sync_copy(zm_hbm, mrow_v)
        for j in range(nzc):
            pltpu.sync_copy(mrow_v, accm.at[pl.ds(row0 + j * _CH, _CH)])
        plsc.subcore_barrier()

        wid = sid * _NC + c
        base = wid * ew_per

        def body(i, carry):
            off = base + i * _CH
            pltpu.sync_copy(src_hbm.at[pl.ds(off, _CH)], idx_v)
            pltpu.sync_copy(m_hbm.at[pl.ds(off, _CH)], mrow_v)
            pltpu.sync_copy(mrow_v, accm.at[idx_v], add=True)
            return carry

        lax.fori_loop(0, niter, body, 0)
        plsc.subcore_barrier()

        for j in range(nzc):
            r0 = row0 + j * _CH
            pltpu.sync_copy(accm.at[pl.ds(r0, _CH)], mrow_v)
            pltpu.sync_copy(mrow_v, outm_hbm.at[c, pl.ds(r0, _CH)])
        plsc.subcore_barrier()

    return sc_scatter


def kernel(x, h, edge_src, edge_dst, edges_r_ij, edges_e_ij,
           W1, b1, W2, b2, Wp1, bp1, Wp2, bp2, W_ih, W_hh, b_ih, b_hh):
    n, f = h.shape
    e = edge_src.shape[0]
    nblk = 2000
    eblk = 2560
    assert n % nblk == 0 and e % eblk == 0 and e % (_NW * _CH) == 0

    w1a_t = W1[:, :f].T
    w1b_t = W1[:, f:2 * f].T
    w1c = W1[:, 2 * f].reshape(1, f)
    b1_r = b1.reshape(1, f)
    w2_t = W2.T
    b2_r = b2.reshape(1, -1)
    wp1_t = Wp1.T
    bp1_r = bp1.reshape(1, -1)
    wp2_t = Wp2.T
    bp2_r = bp2.reshape(1, 1)
    wih_t = W_ih.T
    whh_t = W_hh.T
    bih_r = b_ih.reshape(1, -1)
    bhh_r = b_hh.reshape(1, -1)

    grid_n = n // nblk
    ga, gb = pl.pallas_call(
        _proj_body,
        grid=(grid_n,),
        in_specs=[
            pl.BlockSpec((nblk, f), lambda i: (i, 0)),
            pl.BlockSpec((f, f), lambda i: (0, 0)),
            pl.BlockSpec((f, f), lambda i: (0, 0)),
        ],
        out_specs=[
            pl.BlockSpec((nblk, f), lambda i: (i, 0)),
            pl.BlockSpec((nblk, f), lambda i: (i, 0)),
        ],
        out_shape=[
            jax.ShapeDtypeStruct((n, f), F32),
            jax.ShapeDtypeStruct((n, f), F32),
        ],
    )(h, w1a_t, w1b_t)

    a_g, b_g = _make_sc_gather(n, e, f)(ga, gb, edge_src, edge_dst)

    grid_e = e // eblk
    r_e = edges_r_ij.reshape(e, 1)
    m_e, w_e = pl.pallas_call(
        _edge_body,
        grid=(grid_e,),
        in_specs=[
            pl.BlockSpec((eblk, f), lambda i: (i, 0)),
            pl.BlockSpec((eblk, f), lambda i: (i, 0)),
            pl.BlockSpec((eblk, 1), lambda i: (i, 0)),
            pl.BlockSpec((1, f), lambda i: (0, 0)),
            pl.BlockSpec((1, f), lambda i: (0, 0)),
            pl.BlockSpec((f, f), lambda i: (0, 0)),
            pl.BlockSpec((1, f), lambda i: (0, 0)),
            pl.BlockSpec((f, f), lambda i: (0, 0)),
            pl.BlockSpec((1, f), lambda i: (0, 0)),
            pl.BlockSpec((f, 1), lambda i: (0, 0)),
            pl.BlockSpec((1, 1), lambda i: (0, 0)),
        ],
        out_specs=[
            pl.BlockSpec((eblk, f), lambda i: (i, 0)),
            pl.BlockSpec((eblk, 1), lambda i: (i, 0)),
        ],
        out_shape=[
            jax.ShapeDtypeStruct((e, f), F32),
            jax.ShapeDtypeStruct((e, 1), F32),
        ],
    )(a_g, b_g, r_e, w1c, b1_r, w2_t, b2_r, wp1_t, bp1_r, wp2_t, bp2_r)

    x_ij = x[edge_src] + edges_e_ij * w_e
    sums_x = jax.ops.segment_sum(x_ij, edge_src, num_segments=n)
    cnt = jax.ops.segment_sum(jnp.ones((e,), x_ij.dtype), edge_src,
                              num_segments=n)
    x_prime = sums_x / jnp.maximum(cnt, 1.0)[:, None]
    x_out = jnp.mod(x_prime, 1.0)
    x_diff = x_prime - x

    npt = 128 * (-(-n // (_NS * 128)))
    n_pad = _NS * npt
    zm = jnp.zeros((_CH, f), F32)
    pm = _make_sc_scatter(n_pad, e, f)(m_e, edge_src, zm)

    cnt_col = cnt.reshape(n, 1)
    hp = pl.pallas_call(
        _node_body,
        grid=(grid_n,),
        in_specs=[
            pl.BlockSpec((_NC, nblk, f), lambda i: (0, i, 0)),
            pl.BlockSpec((nblk, 1), lambda i: (i, 0)),
            pl.BlockSpec((nblk, f), lambda i: (i, 0)),
            pl.BlockSpec((f, 3 * f), lambda i: (0, 0)),
            pl.BlockSpec((f, 3 * f), lambda i: (0, 0)),
            pl.BlockSpec((1, 3 * f), lambda i: (0, 0)),
            pl.BlockSpec((1, 3 * f), lambda i: (0, 0)),
        ],
        out_specs=pl.BlockSpec((nblk, f), lambda i: (i, 0)),
        out_shape=jax.ShapeDtypeStruct((n, f), F32),
    )(pm, cnt_col, h, wih_t, whh_t, bih_r, bhh_r)

    return (x_out, x_diff, hp)
